# Initial kernel scaffold; baseline (speedup 1.0000x reference)
import jax
import jax.numpy as jnp
from jax.experimental import pallas as pl


def kernel(h, edge_index, W_embed, b_embed, W0, b0, gamma0, beta0, W1, b1, gamma1, beta1, W_read, b_read):
    raise NotImplementedError("write your pallas kernel here")


# speedup vs baseline: 11.1636x; 11.1636x over previous
import functools

import jax
import jax.numpy as jnp
from jax import lax
from jax.experimental import pallas as pl
from jax.experimental.pallas import tpu as pltpu
from jax.experimental.pallas import tpu_sc as plsc

N = 10000
E = 320000
D_IN = 128
H = 96
HP = 128
C = 64
BN_EPS = 1e-5

NC = 2
NS = 16
NW = NC * NS
LANE_WIN = 128
NWIN = 80
KSTAGE = 40
E_PAD = NW * NWIN * LANE_WIN
PAD = E_PAD - E
NPAD = 10240
ZROWS = NPAD // NS

ROW_BLK = 2000


@functools.partial(
    pl.kernel,
    out_type=(
        jax.ShapeDtypeStruct((NPAD, HP), jnp.float32),
        jax.ShapeDtypeStruct((NPAD, HP), jnp.float32),
    ),
    mesh=plsc.VectorSubcoreMesh(core_axis_name="c", subcore_axis_name="s"),
    scratch_types=[
        pltpu.VMEM((KSTAGE, LANE_WIN), jnp.int32),
        pltpu.VMEM((KSTAGE, LANE_WIN), jnp.int32),
        pltpu.VMEM((LANE_WIN, HP), jnp.float32),
        pltpu.VMEM((LANE_WIN, HP), jnp.float32),
        pltpu.VMEM_SHARED((NPAD, HP), jnp.float32),
        pltpu.SemaphoreType.DMA,
        pltpu.SemaphoreType.DMA,
    ],
)
def _sc_agg(x_hbm, src_hbm, dst_hbm, zero_hbm, out0_hbm, out1_hbm,
            src_v, dst_v, rows0, rows1, agg_sh, sem0, sem1):
    c = lax.axis_index("c")
    s = lax.axis_index("s")
    wid = s * NC + c

    pltpu.sync_copy(zero_hbm, agg_sh.at[pl.ds(s * ZROWS, ZROWS)])
    base = wid * NWIN
    plsc.subcore_barrier()

    def body(i, carry):
        w0 = 2 * i
        pltpu.async_copy(x_hbm.at[src_v.at[w0 + 1]], rows1, sem1)
        pltpu.make_async_copy(x_hbm.at[src_v.at[w0]], rows0, sem0).wait()
        pltpu.sync_copy(rows0, agg_sh.at[dst_v.at[w0]], add=True)
        wn = lax.min(w0 + 2, KSTAGE - 1)
        pltpu.async_copy(x_hbm.at[src_v.at[wn]], rows0, sem0)
        pltpu.make_async_copy(x_hbm.at[src_v.at[w0 + 1]], rows1, sem1).wait()
        pltpu.sync_copy(rows1, agg_sh.at[dst_v.at[w0 + 1]], add=True)
        return carry

    for half in range(NWIN // KSTAGE):
        pltpu.sync_copy(src_hbm.at[pl.ds(base + half * KSTAGE, KSTAGE)], src_v)
        pltpu.sync_copy(dst_hbm.at[pl.ds(base + half * KSTAGE, KSTAGE)], dst_v)
        pltpu.async_copy(x_hbm.at[src_v.at[0]], rows0, sem0)
        lax.fori_loop(0, KSTAGE // 2, body, 0)
        pltpu.make_async_copy(x_hbm.at[src_v.at[0]], rows0, sem0).wait()
    plsc.subcore_barrier()

    @pl.when(c == 0)
    def _():
        pltpu.sync_copy(agg_sh.at[pl.ds(s * ZROWS, ZROWS)],
                        out0_hbm.at[pl.ds(s * ZROWS, ZROWS)])

    @pl.when(c == 1)
    def _():
        pltpu.sync_copy(agg_sh.at[pl.ds(s * ZROWS, ZROWS)],
                        out1_hbm.at[pl.ds(s * ZROWS, ZROWS)])


def _embed_body(h_ref, w_ref, b_ref, o_ref):
    o_ref[...] = (
        jnp.dot(h_ref[...], w_ref[...], preferred_element_type=jnp.float32)
        + b_ref[...]
    )


def _embed(h, W_embed, b_embed):
    return pl.pallas_call(
        _embed_body,
        grid=(N // ROW_BLK,),
        in_specs=[
            pl.BlockSpec((ROW_BLK, D_IN), lambda i: (i, 0)),
            pl.BlockSpec((D_IN, HP), lambda i: (0, 0)),
            pl.BlockSpec((1, HP), lambda i: (0, 0)),
        ],
        out_specs=pl.BlockSpec((ROW_BLK, HP), lambda i: (i, 0)),
        out_shape=jax.ShapeDtypeStruct((N, HP), jnp.float32),
    )(h, W_embed, b_embed)


def _layer_body(x_ref, p0_ref, p1_ref, w_ref, b_ref, g_ref, bt_ref, o_ref):
    y = x_ref[...] + p0_ref[...] + p1_ref[...]
    z = jnp.dot(y, w_ref[...], preferred_element_type=jnp.float32) + b_ref[...]
    scale = g_ref[...] * jax.lax.rsqrt(jnp.float32(1.0 + BN_EPS))
    o_ref[...] = jnp.maximum(z * scale + bt_ref[...], 0.0)


def _layer(x, p0, p1, W, b, gamma, beta):
    return pl.pallas_call(
        _layer_body,
        grid=(N // ROW_BLK,),
        in_specs=[
            pl.BlockSpec((ROW_BLK, HP), lambda i: (i, 0)),
            pl.BlockSpec((ROW_BLK, HP), lambda i: (i, 0)),
            pl.BlockSpec((ROW_BLK, HP), lambda i: (i, 0)),
            pl.BlockSpec((HP, HP), lambda i: (0, 0)),
            pl.BlockSpec((1, HP), lambda i: (0, 0)),
            pl.BlockSpec((1, HP), lambda i: (0, 0)),
            pl.BlockSpec((1, HP), lambda i: (0, 0)),
        ],
        out_specs=pl.BlockSpec((ROW_BLK, HP), lambda i: (i, 0)),
        out_shape=jax.ShapeDtypeStruct((N, HP), jnp.float32),
    )(x, p0, p1, W, b, gamma, beta)


def _final_body(x_ref, p0_ref, p1_ref, w_ref, b_ref, g_ref, bt_ref,
                wr_ref, br_ref, o_ref):
    y = x_ref[...] + p0_ref[...] + p1_ref[...]
    z = jnp.dot(y, w_ref[...], preferred_element_type=jnp.float32) + b_ref[...]
    scale = g_ref[...] * jax.lax.rsqrt(jnp.float32(1.0 + BN_EPS))
    a = jnp.maximum(z * scale + bt_ref[...], 0.0)
    logits = (
        jnp.dot(a, wr_ref[...], preferred_element_type=jnp.float32)
        + br_ref[...]
    )
    m = jnp.max(logits, axis=1, keepdims=True)
    sh = logits - m
    lse = jnp.log(jnp.sum(jnp.exp(sh), axis=1, keepdims=True))
    o_ref[...] = sh - lse


def _final(x, p0, p1, W, b, gamma, beta, W_read, b_read):
    return pl.pallas_call(
        _final_body,
        grid=(N // ROW_BLK,),
        in_specs=[
            pl.BlockSpec((ROW_BLK, HP), lambda i: (i, 0)),
            pl.BlockSpec((ROW_BLK, HP), lambda i: (i, 0)),
            pl.BlockSpec((ROW_BLK, HP), lambda i: (i, 0)),
            pl.BlockSpec((HP, HP), lambda i: (0, 0)),
            pl.BlockSpec((1, HP), lambda i: (0, 0)),
            pl.BlockSpec((1, HP), lambda i: (0, 0)),
            pl.BlockSpec((1, HP), lambda i: (0, 0)),
            pl.BlockSpec((HP, C), lambda i: (0, 0)),
            pl.BlockSpec((1, C), lambda i: (0, 0)),
        ],
        out_specs=pl.BlockSpec((ROW_BLK, C), lambda i: (i, 0)),
        out_shape=jax.ShapeDtypeStruct((N, C), jnp.float32),
    )(x, p0, p1, W, b, gamma, beta, W_read, b_read)


def _pad_h(v):
    return jnp.pad(v, [(0, 0)] * (v.ndim - 1) + [(0, HP - H)])


def kernel(h, edge_index, W_embed, b_embed, W0, b0, gamma0, beta0,
           W1, b1, gamma1, beta1, W_read, b_read):
    src = edge_index[0]
    dst = edge_index[1]
    pad_ids = jnp.arange(PAD, dtype=jnp.int32)
    src_p = jnp.concatenate([src, (pad_ids * 37) % N]).reshape(NW * NWIN, LANE_WIN)
    dst_p = jnp.concatenate([dst, N + (pad_ids % (NPAD - N))]).reshape(
        NW * NWIN, LANE_WIN)
    zeros_blk = jnp.zeros((ZROWS, HP), jnp.float32)

    We = _pad_h(W_embed)
    be = _pad_h(b_embed.reshape(1, H))
    W0p = _pad_h(jnp.pad(W0, [(0, HP - H), (0, 0)]))
    W1p = _pad_h(jnp.pad(W1, [(0, HP - H), (0, 0)]))
    b0p = _pad_h(b0.reshape(1, H))
    b1p = _pad_h(b1.reshape(1, H))
    g0p = _pad_h(gamma0.reshape(1, H))
    g1p = _pad_h(gamma1.reshape(1, H))
    t0p = _pad_h(beta0.reshape(1, H))
    t1p = _pad_h(beta1.reshape(1, H))
    Wr = jnp.pad(W_read, [(0, HP - H), (0, 0)])
    br = b_read.reshape(1, C)

    x = _embed(h, We, be)
    a0, a1 = _sc_agg(x, src_p, dst_p, zeros_blk)
    x = _layer(x, a0, a1, W0p, b0p, g0p, t0p)
    a0, a1 = _sc_agg(x, src_p, dst_p, zeros_blk)
    return _final(x, a0, a1, W1p, b1p, g1p, t1p, Wr, br)


# speedup vs baseline: 11.7389x; 1.0515x over previous
import functools

import jax
import jax.numpy as jnp
from jax import lax
from jax.experimental import pallas as pl
from jax.experimental.pallas import tpu as pltpu
from jax.experimental.pallas import tpu_sc as plsc

N = 10000
E = 320000
D_IN = 128
H = 96
C = 64
BN_EPS = 1e-5

NC = 2
NS = 16
NW = NC * NS
LANE_WIN = 128
NWIN = 80
KSTAGE = 40
E_PAD = NW * NWIN * LANE_WIN
PAD = E_PAD - E
NPAD = 10240
ZROWS = NPAD // NS

ROW_BLK = 2000


@functools.partial(
    pl.kernel,
    out_type=(
        jax.ShapeDtypeStruct((NPAD, H), jnp.float32),
        jax.ShapeDtypeStruct((NPAD, H), jnp.float32),
    ),
    mesh=plsc.VectorSubcoreMesh(core_axis_name="c", subcore_axis_name="s"),
    compiler_params=pltpu.CompilerParams(use_tc_tiling_on_sc=False),
    scratch_types=[
        pltpu.VMEM((KSTAGE, LANE_WIN), jnp.int32),
        pltpu.VMEM((KSTAGE, LANE_WIN), jnp.int32),
        [pltpu.VMEM((LANE_WIN, H), jnp.float32) for _ in range(4)],
        pltpu.VMEM_SHARED((NPAD, H), jnp.float32),
        [pltpu.SemaphoreType.DMA for _ in range(4)],
        [pltpu.SemaphoreType.DMA for _ in range(4)],
    ],
)
def _sc_agg(x_hbm, src_hbm, dst_hbm, zero_hbm, out0_hbm, out1_hbm,
            src_v, dst_v, rows, agg_sh, gsem, ssem):
    c = lax.axis_index("c")
    s = lax.axis_index("s")
    wid = s * NC + c

    def fire_gather(w, b):
        pltpu.async_copy(x_hbm.at[src_v.at[w]], rows[b], gsem[b])

    def wait_gather(b):
        pltpu.make_async_copy(x_hbm.at[src_v.at[0]], rows[b], gsem[b]).wait()

    def fire_scatter(w, b):
        pltpu.async_copy(rows[b], agg_sh.at[dst_v.at[w]], ssem[b], add=True)

    def wait_scatter(b):
        pltpu.make_async_copy(rows[b], agg_sh.at[dst_v.at[0]], ssem[b]).wait()

    pltpu.sync_copy(zero_hbm, agg_sh.at[pl.ds(s * ZROWS, ZROWS)])
    base = wid * NWIN
    plsc.subcore_barrier()

    for half in range(NWIN // KSTAGE):
        pltpu.sync_copy(src_hbm.at[pl.ds(base + half * KSTAGE, KSTAGE)], src_v)
        pltpu.sync_copy(dst_hbm.at[pl.ds(base + half * KSTAGE, KSTAGE)], dst_v)
        fire_gather(0, 0)
        fire_gather(1, 1)
        wait_gather(0)
        fire_scatter(0, 0)
        fire_gather(2, 2)
        wait_gather(1)
        fire_scatter(1, 1)
        fire_gather(3, 3)

        def body(i, carry):
            for b4 in range(4):
                w = 2 + 4 * i + b4
                b = (2 + b4) % 4
                wait_gather(b)
                fire_scatter(w, b)
                wait_scatter((b + 2) % 4)
                fire_gather(w + 2, (b + 2) % 4)
            return carry

        lax.fori_loop(0, (KSTAGE - 4) // 4, body, 0)

        for w in (KSTAGE - 2, KSTAGE - 1):
            b = w % 4
            wait_gather(b)
            fire_scatter(w, b)
            wait_scatter((b + 2) % 4)
        wait_scatter((KSTAGE - 2) % 4)
        wait_scatter((KSTAGE - 1) % 4)

    plsc.subcore_barrier()

    @pl.when(c == 0)
    def _():
        pltpu.sync_copy(agg_sh.at[pl.ds(s * ZROWS, ZROWS)],
                        out0_hbm.at[pl.ds(s * ZROWS, ZROWS)])

    @pl.when(c == 1)
    def _():
        pltpu.sync_copy(agg_sh.at[pl.ds(s * ZROWS, ZROWS)],
                        out1_hbm.at[pl.ds(s * ZROWS, ZROWS)])


def _embed_body(h_ref, w_ref, b_ref, o_ref):
    o_ref[...] = (
        jnp.dot(h_ref[...], w_ref[...], preferred_element_type=jnp.float32)
        + b_ref[...]
    )


def _embed(h, W_embed, b_embed):
    return pl.pallas_call(
        _embed_body,
        grid=(N // ROW_BLK,),
        in_specs=[
            pl.BlockSpec((ROW_BLK, D_IN), lambda i: (i, 0)),
            pl.BlockSpec((D_IN, H), lambda i: (0, 0)),
            pl.BlockSpec((1, H), lambda i: (0, 0)),
        ],
        out_specs=pl.BlockSpec((ROW_BLK, H), lambda i: (i, 0)),
        out_shape=jax.ShapeDtypeStruct((N, H), jnp.float32),
    )(h, W_embed, b_embed.reshape(1, H))


def _layer_body(x_ref, p0_ref, p1_ref, w_ref, b_ref, g_ref, bt_ref, o_ref):
    y = x_ref[...] + p0_ref[...] + p1_ref[...]
    z = jnp.dot(y, w_ref[...], preferred_element_type=jnp.float32) + b_ref[...]
    scale = g_ref[...] * jax.lax.rsqrt(jnp.float32(1.0 + BN_EPS))
    o_ref[...] = jnp.maximum(z * scale + bt_ref[...], 0.0)


def _layer(x, p0, p1, W, b, gamma, beta):
    return pl.pallas_call(
        _layer_body,
        grid=(N // ROW_BLK,),
        in_specs=[
            pl.BlockSpec((ROW_BLK, H), lambda i: (i, 0)),
            pl.BlockSpec((ROW_BLK, H), lambda i: (i, 0)),
            pl.BlockSpec((ROW_BLK, H), lambda i: (i, 0)),
            pl.BlockSpec((H, H), lambda i: (0, 0)),
            pl.BlockSpec((1, H), lambda i: (0, 0)),
            pl.BlockSpec((1, H), lambda i: (0, 0)),
            pl.BlockSpec((1, H), lambda i: (0, 0)),
        ],
        out_specs=pl.BlockSpec((ROW_BLK, H), lambda i: (i, 0)),
        out_shape=jax.ShapeDtypeStruct((N, H), jnp.float32),
    )(x, p0, p1, W, b.reshape(1, H), gamma.reshape(1, H), beta.reshape(1, H))


def _final_body(x_ref, p0_ref, p1_ref, w_ref, b_ref, g_ref, bt_ref,
                wr_ref, br_ref, o_ref):
    y = x_ref[...] + p0_ref[...] + p1_ref[...]
    z = jnp.dot(y, w_ref[...], preferred_element_type=jnp.float32) + b_ref[...]
    scale = g_ref[...] * jax.lax.rsqrt(jnp.float32(1.0 + BN_EPS))
    a = jnp.maximum(z * scale + bt_ref[...], 0.0)
    logits = (
        jnp.dot(a, wr_ref[...], preferred_element_type=jnp.float32)
        + br_ref[...]
    )
    m = jnp.max(logits, axis=1, keepdims=True)
    sh = logits - m
    lse = jnp.log(jnp.sum(jnp.exp(sh), axis=1, keepdims=True))
    o_ref[...] = sh - lse


def _final(x, p0, p1, W, b, gamma, beta, W_read, b_read):
    return pl.pallas_call(
        _final_body,
        grid=(N // ROW_BLK,),
        in_specs=[
            pl.BlockSpec((ROW_BLK, H), lambda i: (i, 0)),
            pl.BlockSpec((ROW_BLK, H), lambda i: (i, 0)),
            pl.BlockSpec((ROW_BLK, H), lambda i: (i, 0)),
            pl.BlockSpec((H, H), lambda i: (0, 0)),
            pl.BlockSpec((1, H), lambda i: (0, 0)),
            pl.BlockSpec((1, H), lambda i: (0, 0)),
            pl.BlockSpec((1, H), lambda i: (0, 0)),
            pl.BlockSpec((H, C), lambda i: (0, 0)),
            pl.BlockSpec((1, C), lambda i: (0, 0)),
        ],
        out_specs=pl.BlockSpec((ROW_BLK, C), lambda i: (i, 0)),
        out_shape=jax.ShapeDtypeStruct((N, C), jnp.float32),
    )(x, p0, p1, W, b.reshape(1, H), gamma.reshape(1, H), beta.reshape(1, H),
      W_read, b_read.reshape(1, C))


def kernel(h, edge_index, W_embed, b_embed, W0, b0, gamma0, beta0,
           W1, b1, gamma1, beta1, W_read, b_read):
    src = edge_index[0]
    dst = edge_index[1]
    pad_ids = jnp.arange(PAD, dtype=jnp.int32)
    src_p = jnp.concatenate([src, (pad_ids * 37) % N]).reshape(NW * NWIN, LANE_WIN)
    dst_p = jnp.concatenate([dst, N + (pad_ids % (NPAD - N))]).reshape(
        NW * NWIN, LANE_WIN)
    zeros_blk = jnp.zeros((ZROWS, H), jnp.float32)

    x = _embed(h, W_embed, b_embed)
    a0, a1 = _sc_agg(x, src_p, dst_p, zeros_blk)
    x = _layer(x, a0, a1, W0, b0, gamma0, beta0)
    a0, a1 = _sc_agg(x, src_p, dst_p, zeros_blk)
    return _final(x, a0, a1, W1, b1, gamma1, beta1, W_read, b_read)


# speedup vs baseline: 14.3735x; 1.2244x over previous
import functools

import jax
import jax.numpy as jnp
from jax import lax
from jax.experimental import pallas as pl
from jax.experimental.pallas import tpu as pltpu
from jax.experimental.pallas import tpu_sc as plsc

N = 10000
E = 320000
D_IN = 128
H = 96
C = 64
BN_EPS = 1e-5

NC = 2
NS = 16
NW = NC * NS
LANE_WIN = 128
NWIN = 80
E_PAD = NW * NWIN * LANE_WIN
PAD = E_PAD - E
NPAD = 10240
ZROWS = NPAD // NS

ROW_BLK = 2000


@functools.partial(
    pl.kernel,
    out_type=(
        jax.ShapeDtypeStruct((NPAD, H), jnp.bfloat16),
        jax.ShapeDtypeStruct((NPAD, H), jnp.bfloat16),
    ),
    mesh=plsc.VectorSubcoreMesh(core_axis_name="c", subcore_axis_name="s"),
    compiler_params=pltpu.CompilerParams(use_tc_tiling_on_sc=False),
    scratch_types=[
        pltpu.VMEM((NWIN, LANE_WIN), jnp.int32),
        pltpu.VMEM((NWIN, LANE_WIN), jnp.int32),
        [pltpu.VMEM((LANE_WIN, H), jnp.bfloat16) for _ in range(4)],
        pltpu.VMEM_SHARED((NPAD, H), jnp.bfloat16),
        [pltpu.SemaphoreType.DMA for _ in range(4)],
        [pltpu.SemaphoreType.DMA for _ in range(4)],
    ],
)
def _sc_agg(x_hbm, src_hbm, dst_hbm, zero_hbm, out0_hbm, out1_hbm,
            src_v, dst_v, rows, agg_sh, gsem, ssem):
    c = lax.axis_index("c")
    s = lax.axis_index("s")
    wid = s * NC + c

    def fire_gather(w, b):
        pltpu.async_copy(x_hbm.at[src_v.at[w]], rows[b], gsem[b])

    def wait_gather(b):
        pltpu.make_async_copy(x_hbm.at[src_v.at[0]], rows[b], gsem[b]).wait()

    def fire_scatter(w, b):
        pltpu.async_copy(rows[b], agg_sh.at[dst_v.at[w]], ssem[b], add=True)

    def wait_scatter(b):
        pltpu.make_async_copy(rows[b], agg_sh.at[dst_v.at[0]], ssem[b]).wait()

    pltpu.sync_copy(zero_hbm, agg_sh.at[pl.ds(s * ZROWS, ZROWS)])
    base = wid * NWIN
    pltpu.sync_copy(src_hbm.at[pl.ds(base, NWIN)], src_v)
    pltpu.sync_copy(dst_hbm.at[pl.ds(base, NWIN)], dst_v)
    plsc.subcore_barrier()

    fire_gather(0, 0)
    fire_gather(1, 1)
    wait_gather(0)
    fire_scatter(0, 0)
    fire_gather(2, 2)
    wait_gather(1)
    fire_scatter(1, 1)
    fire_gather(3, 3)

    def body(i, carry):
        for b4 in range(4):
            w = 2 + 4 * i + b4
            b = (2 + b4) % 4
            wait_gather(b)
            fire_scatter(w, b)
            wait_scatter((b + 2) % 4)
            fire_gather(w + 2, (b + 2) % 4)
        return carry

    lax.fori_loop(0, (NWIN - 4) // 4, body, 0)

    for w in (NWIN - 2, NWIN - 1):
        b = w % 4
        wait_gather(b)
        fire_scatter(w, b)
        wait_scatter((b + 2) % 4)
    wait_scatter((NWIN - 2) % 4)
    wait_scatter((NWIN - 1) % 4)

    plsc.subcore_barrier()

    @pl.when(c == 0)
    def _():
        pltpu.sync_copy(agg_sh.at[pl.ds(s * ZROWS, ZROWS)],
                        out0_hbm.at[pl.ds(s * ZROWS, ZROWS)])

    @pl.when(c == 1)
    def _():
        pltpu.sync_copy(agg_sh.at[pl.ds(s * ZROWS, ZROWS)],
                        out1_hbm.at[pl.ds(s * ZROWS, ZROWS)])


def _embed_body(h_ref, w_ref, b_ref, o_ref, ob_ref):
    z = (
        jnp.dot(h_ref[...], w_ref[...], preferred_element_type=jnp.float32)
        + b_ref[...]
    )
    o_ref[...] = z
    ob_ref[...] = z.astype(jnp.bfloat16)


def _embed(h, W_embed, b_embed):
    return pl.pallas_call(
        _embed_body,
        grid=(N // ROW_BLK,),
        in_specs=[
            pl.BlockSpec((ROW_BLK, D_IN), lambda i: (i, 0)),
            pl.BlockSpec((D_IN, H), lambda i: (0, 0)),
            pl.BlockSpec((1, H), lambda i: (0, 0)),
        ],
        out_specs=[
            pl.BlockSpec((ROW_BLK, H), lambda i: (i, 0)),
            pl.BlockSpec((ROW_BLK, H), lambda i: (i, 0)),
        ],
        out_shape=[
            jax.ShapeDtypeStruct((N, H), jnp.float32),
            jax.ShapeDtypeStruct((N, H), jnp.bfloat16),
        ],
    )(h, W_embed, b_embed.reshape(1, H))


def _layer_body(x_ref, p0_ref, p1_ref, w_ref, b_ref, g_ref, bt_ref,
                o_ref, ob_ref):
    y = (x_ref[...]
         + p0_ref[...].astype(jnp.float32)
         + p1_ref[...].astype(jnp.float32))
    z = jnp.dot(y, w_ref[...], preferred_element_type=jnp.float32) + b_ref[...]
    scale = g_ref[...] * jax.lax.rsqrt(jnp.float32(1.0 + BN_EPS))
    a = jnp.maximum(z * scale + bt_ref[...], 0.0)
    o_ref[...] = a
    ob_ref[...] = a.astype(jnp.bfloat16)


def _layer(x, p0, p1, W, b, gamma, beta):
    return pl.pallas_call(
        _layer_body,
        grid=(N // ROW_BLK,),
        in_specs=[
            pl.BlockSpec((ROW_BLK, H), lambda i: (i, 0)),
            pl.BlockSpec((ROW_BLK, H), lambda i: (i, 0)),
            pl.BlockSpec((ROW_BLK, H), lambda i: (i, 0)),
            pl.BlockSpec((H, H), lambda i: (0, 0)),
            pl.BlockSpec((1, H), lambda i: (0, 0)),
            pl.BlockSpec((1, H), lambda i: (0, 0)),
            pl.BlockSpec((1, H), lambda i: (0, 0)),
        ],
        out_specs=[
            pl.BlockSpec((ROW_BLK, H), lambda i: (i, 0)),
            pl.BlockSpec((ROW_BLK, H), lambda i: (i, 0)),
        ],
        out_shape=[
            jax.ShapeDtypeStruct((N, H), jnp.float32),
            jax.ShapeDtypeStruct((N, H), jnp.bfloat16),
        ],
    )(x, p0, p1, W, b.reshape(1, H), gamma.reshape(1, H), beta.reshape(1, H))


def _final_body(x_ref, p0_ref, p1_ref, w_ref, b_ref, g_ref, bt_ref,
                wr_ref, br_ref, o_ref):
    y = (x_ref[...]
         + p0_ref[...].astype(jnp.float32)
         + p1_ref[...].astype(jnp.float32))
    z = jnp.dot(y, w_ref[...], preferred_element_type=jnp.float32) + b_ref[...]
    scale = g_ref[...] * jax.lax.rsqrt(jnp.float32(1.0 + BN_EPS))
    a = jnp.maximum(z * scale + bt_ref[...], 0.0)
    logits = (
        jnp.dot(a, wr_ref[...], preferred_element_type=jnp.float32)
        + br_ref[...]
    )
    m = jnp.max(logits, axis=1, keepdims=True)
    sh = logits - m
    lse = jnp.log(jnp.sum(jnp.exp(sh), axis=1, keepdims=True))
    o_ref[...] = sh - lse


def _final(x, p0, p1, W, b, gamma, beta, W_read, b_read):
    return pl.pallas_call(
        _final_body,
        grid=(N // ROW_BLK,),
        in_specs=[
            pl.BlockSpec((ROW_BLK, H), lambda i: (i, 0)),
            pl.BlockSpec((ROW_BLK, H), lambda i: (i, 0)),
            pl.BlockSpec((ROW_BLK, H), lambda i: (i, 0)),
            pl.BlockSpec((H, H), lambda i: (0, 0)),
            pl.BlockSpec((1, H), lambda i: (0, 0)),
            pl.BlockSpec((1, H), lambda i: (0, 0)),
            pl.BlockSpec((1, H), lambda i: (0, 0)),
            pl.BlockSpec((H, C), lambda i: (0, 0)),
            pl.BlockSpec((1, C), lambda i: (0, 0)),
        ],
        out_specs=pl.BlockSpec((ROW_BLK, C), lambda i: (i, 0)),
        out_shape=jax.ShapeDtypeStruct((N, C), jnp.float32),
    )(x, p0, p1, W, b.reshape(1, H), gamma.reshape(1, H), beta.reshape(1, H),
      W_read, b_read.reshape(1, C))


def kernel(h, edge_index, W_embed, b_embed, W0, b0, gamma0, beta0,
           W1, b1, gamma1, beta1, W_read, b_read):
    src = edge_index[0]
    dst = edge_index[1]
    pad_ids = jnp.arange(PAD, dtype=jnp.int32)
    src_p = jnp.concatenate([src, (pad_ids * 37) % N]).reshape(NW * NWIN, LANE_WIN)
    dst_p = jnp.concatenate([dst, N + (pad_ids % (NPAD - N))]).reshape(
        NW * NWIN, LANE_WIN)
    zeros_blk = jnp.zeros((ZROWS, H), jnp.bfloat16)

    x, xb = _embed(h, W_embed, b_embed)
    a0, a1 = _sc_agg(xb, src_p, dst_p, zeros_blk)
    x, xb = _layer(x, a0, a1, W0, b0, gamma0, beta0)
    a0, a1 = _sc_agg(xb, src_p, dst_p, zeros_blk)
    return _final(x, a0, a1, W1, b1, gamma1, beta1, W_read, b_read)


# speedup vs baseline: 14.5439x; 1.0119x over previous
import functools

import jax
import jax.numpy as jnp
from jax import lax
from jax.experimental import pallas as pl
from jax.experimental.pallas import tpu as pltpu
from jax.experimental.pallas import tpu_sc as plsc

N = 10000
E = 320000
D_IN = 128
H = 96
C = 64
BN_EPS = 1e-5

NC = 2
NS = 16
NW = NC * NS
EPW = E // NW
LANE_WIN = 128
NWIN = EPW // LANE_WIN
TAIL = EPW - NWIN * LANE_WIN
NPAD = 10240
ZROWS = NPAD // NS

ROW_BLK = 2000


@functools.partial(
    pl.kernel,
    out_type=(
        jax.ShapeDtypeStruct((NPAD, H), jnp.bfloat16),
        jax.ShapeDtypeStruct((NPAD, H), jnp.bfloat16),
    ),
    mesh=plsc.VectorSubcoreMesh(core_axis_name="c", subcore_axis_name="s"),
    compiler_params=pltpu.CompilerParams(use_tc_tiling_on_sc=False),
    scratch_types=[
        pltpu.VMEM((EPW,), jnp.int32),
        pltpu.VMEM((EPW,), jnp.int32),
        [pltpu.VMEM((LANE_WIN, H), jnp.bfloat16) for _ in range(4)],
        pltpu.VMEM((TAIL, H), jnp.bfloat16),
        pltpu.VMEM_SHARED((NPAD, H), jnp.bfloat16),
        [pltpu.SemaphoreType.DMA for _ in range(4)],
        [pltpu.SemaphoreType.DMA for _ in range(4)],
    ],
)
def _sc_agg(x_hbm, src_hbm, dst_hbm, zero_hbm, out0_hbm, out1_hbm,
            src_v, dst_v, rows, rows_t, agg_sh, gsem, ssem):
    c = lax.axis_index("c")
    s = lax.axis_index("s")
    wid = s * NC + c

    def fire_gather(w, b):
        pltpu.async_copy(x_hbm.at[src_v.at[pl.ds(w * LANE_WIN, LANE_WIN)]],
                         rows[b], gsem[b])

    def wait_gather(b):
        pltpu.make_async_copy(x_hbm.at[src_v.at[pl.ds(0, LANE_WIN)]],
                              rows[b], gsem[b]).wait()

    def fire_scatter(w, b):
        pltpu.async_copy(rows[b],
                         agg_sh.at[dst_v.at[pl.ds(w * LANE_WIN, LANE_WIN)]],
                         ssem[b], add=True)

    def wait_scatter(b):
        pltpu.make_async_copy(rows[b],
                              agg_sh.at[dst_v.at[pl.ds(0, LANE_WIN)]],
                              ssem[b]).wait()

    pltpu.sync_copy(zero_hbm, agg_sh.at[pl.ds(s * ZROWS, ZROWS)])
    base = wid * EPW
    pltpu.sync_copy(src_hbm.at[pl.ds(base, EPW)], src_v)
    pltpu.sync_copy(dst_hbm.at[pl.ds(base, EPW)], dst_v)
    plsc.subcore_barrier()

    fire_gather(0, 0)
    fire_gather(1, 1)
    wait_gather(0)
    fire_scatter(0, 0)
    fire_gather(2, 2)
    wait_gather(1)
    fire_scatter(1, 1)
    fire_gather(3, 3)

    def body(i, carry):
        for b4 in range(4):
            w = 2 + 4 * i + b4
            b = (2 + b4) % 4
            wait_gather(b)
            fire_scatter(w, b)
            wait_scatter((b + 2) % 4)
            fire_gather(w + 2, (b + 2) % 4)
        return carry

    lax.fori_loop(0, (NWIN - 6) // 4, body, 0)

    for w in range(NWIN - 4, NWIN):
        b = w % 4
        wait_gather(b)
        fire_scatter(w, b)
        wait_scatter((b + 2) % 4)
        if w + 2 < NWIN:
            fire_gather(w + 2, (b + 2) % 4)
    wait_scatter((NWIN - 2) % 4)
    wait_scatter((NWIN - 1) % 4)

    tail_off = NWIN * LANE_WIN
    pltpu.async_copy(x_hbm.at[src_v.at[pl.ds(tail_off, TAIL)]],
                     rows_t, gsem[0])
    pltpu.make_async_copy(x_hbm.at[src_v.at[pl.ds(tail_off, TAIL)]],
                          rows_t, gsem[0]).wait()
    pltpu.sync_copy(rows_t, agg_sh.at[dst_v.at[pl.ds(tail_off, TAIL)]],
                    add=True)

    plsc.subcore_barrier()

    @pl.when(c == 0)
    def _():
        pltpu.sync_copy(agg_sh.at[pl.ds(s * ZROWS, ZROWS)],
                        out0_hbm.at[pl.ds(s * ZROWS, ZROWS)])

    @pl.when(c == 1)
    def _():
        pltpu.sync_copy(agg_sh.at[pl.ds(s * ZROWS, ZROWS)],
                        out1_hbm.at[pl.ds(s * ZROWS, ZROWS)])


def _embed_body(h_ref, w_ref, b_ref, o_ref, ob_ref):
    z = (
        jnp.dot(h_ref[...], w_ref[...], preferred_element_type=jnp.float32)
        + b_ref[...]
    )
    o_ref[...] = z
    ob_ref[...] = z.astype(jnp.bfloat16)


def _embed(h, W_embed, b_embed):
    return pl.pallas_call(
        _embed_body,
        grid=(N // ROW_BLK,),
        in_specs=[
            pl.BlockSpec((ROW_BLK, D_IN), lambda i: (i, 0)),
            pl.BlockSpec((D_IN, H), lambda i: (0, 0)),
            pl.BlockSpec((1, H), lambda i: (0, 0)),
        ],
        out_specs=[
            pl.BlockSpec((ROW_BLK, H), lambda i: (i, 0)),
            pl.BlockSpec((ROW_BLK, H), lambda i: (i, 0)),
        ],
        out_shape=[
            jax.ShapeDtypeStruct((N, H), jnp.float32),
            jax.ShapeDtypeStruct((N, H), jnp.bfloat16),
        ],
    )(h, W_embed, b_embed.reshape(1, H))


def _layer_body(x_ref, p0_ref, p1_ref, w_ref, b_ref, g_ref, bt_ref,
                o_ref, ob_ref):
    y = (x_ref[...]
         + p0_ref[...].astype(jnp.float32)
         + p1_ref[...].astype(jnp.float32))
    z = jnp.dot(y, w_ref[...], preferred_element_type=jnp.float32) + b_ref[...]
    scale = g_ref[...] * jax.lax.rsqrt(jnp.float32(1.0 + BN_EPS))
    a = jnp.maximum(z * scale + bt_ref[...], 0.0)
    o_ref[...] = a
    ob_ref[...] = a.astype(jnp.bfloat16)


def _layer(x, p0, p1, W, b, gamma, beta):
    return pl.pallas_call(
        _layer_body,
        grid=(N // ROW_BLK,),
        in_specs=[
            pl.BlockSpec((ROW_BLK, H), lambda i: (i, 0)),
            pl.BlockSpec((ROW_BLK, H), lambda i: (i, 0)),
            pl.BlockSpec((ROW_BLK, H), lambda i: (i, 0)),
            pl.BlockSpec((H, H), lambda i: (0, 0)),
            pl.BlockSpec((1, H), lambda i: (0, 0)),
            pl.BlockSpec((1, H), lambda i: (0, 0)),
            pl.BlockSpec((1, H), lambda i: (0, 0)),
        ],
        out_specs=[
            pl.BlockSpec((ROW_BLK, H), lambda i: (i, 0)),
            pl.BlockSpec((ROW_BLK, H), lambda i: (i, 0)),
        ],
        out_shape=[
            jax.ShapeDtypeStruct((N, H), jnp.float32),
            jax.ShapeDtypeStruct((N, H), jnp.bfloat16),
        ],
    )(x, p0, p1, W, b.reshape(1, H), gamma.reshape(1, H), beta.reshape(1, H))


def _final_body(x_ref, p0_ref, p1_ref, w_ref, b_ref, g_ref, bt_ref,
                wr_ref, br_ref, o_ref):
    y = (x_ref[...]
         + p0_ref[...].astype(jnp.float32)
         + p1_ref[...].astype(jnp.float32))
    z = jnp.dot(y, w_ref[...], preferred_element_type=jnp.float32) + b_ref[...]
    scale = g_ref[...] * jax.lax.rsqrt(jnp.float32(1.0 + BN_EPS))
    a = jnp.maximum(z * scale + bt_ref[...], 0.0)
    logits = (
        jnp.dot(a, wr_ref[...], preferred_element_type=jnp.float32)
        + br_ref[...]
    )
    m = jnp.max(logits, axis=1, keepdims=True)
    sh = logits - m
    lse = jnp.log(jnp.sum(jnp.exp(sh), axis=1, keepdims=True))
    o_ref[...] = sh - lse


def _final(x, p0, p1, W, b, gamma, beta, W_read, b_read):
    return pl.pallas_call(
        _final_body,
        grid=(N // ROW_BLK,),
        in_specs=[
            pl.BlockSpec((ROW_BLK, H), lambda i: (i, 0)),
            pl.BlockSpec((ROW_BLK, H), lambda i: (i, 0)),
            pl.BlockSpec((ROW_BLK, H), lambda i: (i, 0)),
            pl.BlockSpec((H, H), lambda i: (0, 0)),
            pl.BlockSpec((1, H), lambda i: (0, 0)),
            pl.BlockSpec((1, H), lambda i: (0, 0)),
            pl.BlockSpec((1, H), lambda i: (0, 0)),
            pl.BlockSpec((H, C), lambda i: (0, 0)),
            pl.BlockSpec((1, C), lambda i: (0, 0)),
        ],
        out_specs=pl.BlockSpec((ROW_BLK, C), lambda i: (i, 0)),
        out_shape=jax.ShapeDtypeStruct((N, C), jnp.float32),
    )(x, p0, p1, W, b.reshape(1, H), gamma.reshape(1, H), beta.reshape(1, H),
      W_read, b_read.reshape(1, C))


def kernel(h, edge_index, W_embed, b_embed, W0, b0, gamma0, beta0,
           W1, b1, gamma1, beta1, W_read, b_read):
    src = edge_index[0]
    dst = edge_index[1]
    zeros_blk = jnp.zeros((ZROWS, H), jnp.bfloat16)

    x, xb = _embed(h, W_embed, b_embed)
    a0, a1 = _sc_agg(xb, src, dst, zeros_blk)
    x, xb = _layer(x, a0, a1, W0, b0, gamma0, beta0)
    a0, a1 = _sc_agg(xb, src, dst, zeros_blk)
    return _final(x, a0, a1, W1, b1, gamma1, beta1, W_read, b_read)


# speedup vs baseline: 16.4743x; 1.1327x over previous
import functools

import jax
import jax.numpy as jnp
from jax import lax
from jax.experimental import pallas as pl
from jax.experimental.pallas import tpu as pltpu
from jax.experimental.pallas import tpu_sc as plsc

N = 10000
E = 320000
D_IN = 128
H = 96
C = 64
BN_EPS = 1e-5

NC = 2
NS = 16
NW = NC * NS
EPW = E // NW
LANE_WIN = 128
NWIN = EPW // LANE_WIN
TAIL = EPW - NWIN * LANE_WIN
NPAD = 10240
ZROWS = NPAD // NS

ROW_BLK = 2000


@functools.partial(
    pl.kernel,
    out_type=(
        jax.ShapeDtypeStruct((NPAD, H), jnp.bfloat16),
        jax.ShapeDtypeStruct((NPAD, H), jnp.bfloat16),
    ),
    mesh=plsc.VectorSubcoreMesh(core_axis_name="c", subcore_axis_name="s"),
    compiler_params=pltpu.CompilerParams(use_tc_tiling_on_sc=False),
    scratch_types=[
        pltpu.VMEM((EPW,), jnp.int32),
        pltpu.VMEM((EPW,), jnp.int32),
        [pltpu.VMEM((LANE_WIN, H), jnp.bfloat16) for _ in range(6)],
        pltpu.VMEM((TAIL, H), jnp.bfloat16),
        pltpu.VMEM_SHARED((NPAD, H), jnp.bfloat16),
        [pltpu.SemaphoreType.DMA for _ in range(6)],
        [pltpu.SemaphoreType.DMA for _ in range(6)],
        pltpu.SemaphoreType.DMA,
    ],
)
def _sc_agg(edge_hbm, x_hbm, zero_hbm, out0_hbm, out1_hbm,
            src_v, dst_v, rows, rows_t, agg_sh, gsem, ssem, stsem):
    c = lax.axis_index("c")
    s = lax.axis_index("s")
    wid = s * NC + c

    def fire_gather(w, b):
        pltpu.async_copy(x_hbm.at[src_v.at[pl.ds(w * LANE_WIN, LANE_WIN)]],
                         rows[b], gsem[b])

    def wait_gather(b):
        pltpu.make_async_copy(x_hbm.at[src_v.at[pl.ds(0, LANE_WIN)]],
                              rows[b], gsem[b]).wait()

    def fire_scatter(w, b):
        pltpu.async_copy(rows[b],
                         agg_sh.at[dst_v.at[pl.ds(w * LANE_WIN, LANE_WIN)]],
                         ssem[b], add=True)

    def wait_scatter(b):
        pltpu.make_async_copy(rows[b],
                              agg_sh.at[dst_v.at[pl.ds(0, LANE_WIN)]],
                              ssem[b]).wait()

    base = wid * EPW
    zcp = pltpu.make_async_copy(zero_hbm, agg_sh.at[pl.ds(s * ZROWS, ZROWS)],
                                stsem)
    zcp.start()
    pltpu.async_copy(edge_hbm.at[0, pl.ds(base, EPW)], src_v, gsem[0])
    pltpu.async_copy(edge_hbm.at[1, pl.ds(base, EPW)], dst_v, gsem[1])
    pltpu.make_async_copy(edge_hbm.at[0, pl.ds(base, EPW)], src_v,
                          gsem[0]).wait()
    pltpu.make_async_copy(edge_hbm.at[1, pl.ds(base, EPW)], dst_v,
                          gsem[1]).wait()
    zcp.wait()
    plsc.subcore_barrier()

    fire_gather(0, 0)
    fire_gather(1, 1)
    fire_gather(2, 2)
    for w in range(3):
        wait_gather(w)
        fire_scatter(w, w)
        fire_gather(w + 3, w + 3)

    def body(i, carry):
        for b6 in range(6):
            w = 3 + 6 * i + b6
            b = (3 + b6) % 6
            wait_gather(b)
            fire_scatter(w, b)
            wait_scatter((b + 3) % 6)
            fire_gather(w + 3, (b + 3) % 6)
        return carry

    lax.fori_loop(0, (NWIN - 6) // 6, body, 0)

    for w in range(NWIN - 3, NWIN):
        b = w % 6
        wait_gather(b)
        fire_scatter(w, b)
        wait_scatter((b + 3) % 6)
    for w in range(NWIN - 3, NWIN):
        wait_scatter(w % 6)

    tail_off = NWIN * LANE_WIN
    pltpu.async_copy(x_hbm.at[src_v.at[pl.ds(tail_off, TAIL)]],
                     rows_t, gsem[0])
    pltpu.make_async_copy(x_hbm.at[src_v.at[pl.ds(tail_off, TAIL)]],
                          rows_t, gsem[0]).wait()
    pltpu.sync_copy(rows_t, agg_sh.at[dst_v.at[pl.ds(tail_off, TAIL)]],
                    add=True)

    plsc.subcore_barrier()

    @pl.when(c == 0)
    def _():
        pltpu.sync_copy(agg_sh.at[pl.ds(s * ZROWS, ZROWS)],
                        out0_hbm.at[pl.ds(s * ZROWS, ZROWS)])

    @pl.when(c == 1)
    def _():
        pltpu.sync_copy(agg_sh.at[pl.ds(s * ZROWS, ZROWS)],
                        out1_hbm.at[pl.ds(s * ZROWS, ZROWS)])


def _embed_body(h_ref, w_ref, b_ref, o_ref, ob_ref):
    z = (
        jnp.dot(h_ref[...], w_ref[...], preferred_element_type=jnp.float32)
        + b_ref[...]
    )
    o_ref[...] = z
    ob_ref[...] = z.astype(jnp.bfloat16)


def _embed(h, W_embed, b_embed):
    return pl.pallas_call(
        _embed_body,
        grid=(N // ROW_BLK,),
        in_specs=[
            pl.BlockSpec((ROW_BLK, D_IN), lambda i: (i, 0)),
            pl.BlockSpec((D_IN, H), lambda i: (0, 0)),
            pl.BlockSpec((1, H), lambda i: (0, 0)),
        ],
        out_specs=[
            pl.BlockSpec((ROW_BLK, H), lambda i: (i, 0)),
            pl.BlockSpec((ROW_BLK, H), lambda i: (i, 0)),
        ],
        out_shape=[
            jax.ShapeDtypeStruct((N, H), jnp.float32),
            jax.ShapeDtypeStruct((N, H), jnp.bfloat16),
        ],
    )(h, W_embed, b_embed.reshape(1, H))


def _layer_body(x_ref, p0_ref, p1_ref, w_ref, b_ref, g_ref, bt_ref,
                o_ref, ob_ref):
    y = (x_ref[...]
         + p0_ref[...].astype(jnp.float32)
         + p1_ref[...].astype(jnp.float32))
    z = jnp.dot(y, w_ref[...], preferred_element_type=jnp.float32) + b_ref[...]
    scale = g_ref[...] * jax.lax.rsqrt(jnp.float32(1.0 + BN_EPS))
    a = jnp.maximum(z * scale + bt_ref[...], 0.0)
    o_ref[...] = a
    ob_ref[...] = a.astype(jnp.bfloat16)


def _layer(x, p0, p1, W, b, gamma, beta):
    return pl.pallas_call(
        _layer_body,
        grid=(N // ROW_BLK,),
        in_specs=[
            pl.BlockSpec((ROW_BLK, H), lambda i: (i, 0)),
            pl.BlockSpec((ROW_BLK, H), lambda i: (i, 0)),
            pl.BlockSpec((ROW_BLK, H), lambda i: (i, 0)),
            pl.BlockSpec((H, H), lambda i: (0, 0)),
            pl.BlockSpec((1, H), lambda i: (0, 0)),
            pl.BlockSpec((1, H), lambda i: (0, 0)),
            pl.BlockSpec((1, H), lambda i: (0, 0)),
        ],
        out_specs=[
            pl.BlockSpec((ROW_BLK, H), lambda i: (i, 0)),
            pl.BlockSpec((ROW_BLK, H), lambda i: (i, 0)),
        ],
        out_shape=[
            jax.ShapeDtypeStruct((N, H), jnp.float32),
            jax.ShapeDtypeStruct((N, H), jnp.bfloat16),
        ],
    )(x, p0, p1, W, b.reshape(1, H), gamma.reshape(1, H), beta.reshape(1, H))


def _final_body(x_ref, p0_ref, p1_ref, w_ref, b_ref, g_ref, bt_ref,
                wr_ref, br_ref, o_ref):
    y = (x_ref[...]
         + p0_ref[...].astype(jnp.float32)
         + p1_ref[...].astype(jnp.float32))
    z = jnp.dot(y, w_ref[...], preferred_element_type=jnp.float32) + b_ref[...]
    scale = g_ref[...] * jax.lax.rsqrt(jnp.float32(1.0 + BN_EPS))
    a = jnp.maximum(z * scale + bt_ref[...], 0.0)
    logits = (
        jnp.dot(a, wr_ref[...], preferred_element_type=jnp.float32)
        + br_ref[...]
    )
    m = jnp.max(logits, axis=1, keepdims=True)
    sh = logits - m
    lse = jnp.log(jnp.sum(jnp.exp(sh), axis=1, keepdims=True))
    o_ref[...] = sh - lse


def _final(x, p0, p1, W, b, gamma, beta, W_read, b_read):
    return pl.pallas_call(
        _final_body,
        grid=(N // ROW_BLK,),
        in_specs=[
            pl.BlockSpec((ROW_BLK, H), lambda i: (i, 0)),
            pl.BlockSpec((ROW_BLK, H), lambda i: (i, 0)),
            pl.BlockSpec((ROW_BLK, H), lambda i: (i, 0)),
            pl.BlockSpec((H, H), lambda i: (0, 0)),
            pl.BlockSpec((1, H), lambda i: (0, 0)),
            pl.BlockSpec((1, H), lambda i: (0, 0)),
            pl.BlockSpec((1, H), lambda i: (0, 0)),
            pl.BlockSpec((H, C), lambda i: (0, 0)),
            pl.BlockSpec((1, C), lambda i: (0, 0)),
        ],
        out_specs=pl.BlockSpec((ROW_BLK, C), lambda i: (i, 0)),
        out_shape=jax.ShapeDtypeStruct((N, C), jnp.float32),
    )(x, p0, p1, W, b.reshape(1, H), gamma.reshape(1, H), beta.reshape(1, H),
      W_read, b_read.reshape(1, C))


def kernel(h, edge_index, W_embed, b_embed, W0, b0, gamma0, beta0,
           W1, b1, gamma1, beta1, W_read, b_read):
    zeros_blk = jnp.zeros((ZROWS, H), jnp.bfloat16)

    x, xb = _embed(h, W_embed, b_embed)
    a0, a1 = _sc_agg(edge_index, xb, zeros_blk)
    x, xb = _layer(x, a0, a1, W0, b0, gamma0, beta0)
    a0, a1 = _sc_agg(edge_index, xb, zeros_blk)
    return _final(x, a0, a1, W1, b1, gamma1, beta1, W_read, b_read)
